# baseline (speedup 1.0000x reference)
import functools

import jax
import jax.numpy as jnp
from jax import lax
from jax.experimental import pallas as pl
from jax.experimental.pallas import tpu as pltpu
from jax.experimental.pallas import tpu_sc as plsc

_N = 10000
_E = 320000
_NW = 32
_CE = 2000
_NCH = _E // _CE
_BLK = _CE // 16

_mesh = plsc.VectorSubcoreMesh(core_axis_name="c", subcore_axis_name="s")


def _make_edge_kernel(cpt):
  @functools.partial(
      pl.kernel,
      mesh=_mesh,
      compiler_params=pltpu.CompilerParams(use_tc_tiling_on_sc=False,
                                           needs_layout_passes=False),
      out_type=jax.ShapeDtypeStruct((_NW, cpt, _N), jnp.float32),
      scratch_types=(
          [pltpu.VMEM((cpt * _N,), jnp.float32)]
          + [pltpu.VMEM((cpt * _N,), jnp.float32)]
          + [pltpu.VMEM((_CE,), jnp.int32)] * 2
          + [pltpu.VMEM((_CE,), jnp.int32)] * 2
          + [pltpu.VMEM((_CE,), jnp.float32)] * 2
          + [pltpu.SemaphoreType.DMA] * 2
      ),
  )
  def ek(t_hbm, src_hbm, dst_hbm, w_hbm, out_hbm,
         tv, av, srcv0, srcv1, dstv0, dstv1, wv0, wv1, sem0, sem1):
    c = lax.axis_index("c")
    s = lax.axis_index("s")
    wid = s * 2 + c

    srcv = (srcv0, srcv1)
    dstv = (dstv0, dstv1)
    wv = (wv0, wv1)
    sem = (sem0, sem1)

    for j in range(cpt):
      pltpu.sync_copy(t_hbm.at[wid, j], tv.at[pl.ds(j * _N, _N)])
    zv = jnp.zeros((16,), jnp.float32)

    def zrow(r, carry):
      av[pl.ds(r * 16, 16)] = zv
      return carry

    lax.fori_loop(0, cpt * _N // 16, zrow, 0)

    def stage_start(ci, b):
      off = ci * _CE
      pltpu.async_copy(src_hbm.at[pl.ds(off, _CE)], srcv[b], sem[b])
      pltpu.async_copy(dst_hbm.at[pl.ds(off, _CE)], dstv[b], sem[b])
      pltpu.async_copy(w_hbm.at[pl.ds(off, _CE)], wv[b], sem[b])

    def stage_wait(ci, b):
      off = ci * _CE
      pltpu.make_async_copy(src_hbm.at[pl.ds(off, _CE)], srcv[b], sem[b]).wait()
      pltpu.make_async_copy(dst_hbm.at[pl.ds(off, _CE)], dstv[b], sem[b]).wait()
      pltpu.make_async_copy(w_hbm.at[pl.ds(off, _CE)], wv[b], sem[b]).wait()

    def process(b):
      def blk(k, carry):
        pos = pl.ds(k * 16, 16)
        s16 = srcv[b][pos]
        d16 = dstv[b][pos]
        w16 = wv[b][pos]
        occ, _ = plsc.scan_count(d16)
        first = occ == 0
        for j in range(cpt):
          vals = plsc.load_gather(tv, [s16 + (j * _N)]) * w16
          plsc.addupdate_scatter(av, [d16 + (j * _N)], vals, mask=first)

        maxo = jnp.max(occ)

        @pl.when(maxo > 0)
        def _():
          def opass(kk, c2):
            m = occ == kk
            for j in range(cpt):
              vals = plsc.load_gather(tv, [s16 + (j * _N)]) * w16
              plsc.addupdate_scatter(av, [d16 + (j * _N)], vals, mask=m)
            return c2

          lax.fori_loop(1, maxo + 1, opass, 0)
        return carry

      lax.fori_loop(0, _BLK, blk, 0)

    stage_start(0, 0)
    stage_wait(0, 0)

    def pair(ip, carry):
      for b in range(2):
        ci = 2 * ip + b
        nb = 1 - b
        stage_start(ci + 1, nb)
        process(b)
        stage_wait(ci + 1, nb)
      return carry

    lax.fori_loop(0, _NCH // 2 - 1, pair, 0)

    stage_start(_NCH - 1, 1)
    process(0)
    stage_wait(_NCH - 1, 1)
    process(1)

    for j in range(cpt):
      pltpu.sync_copy(av.at[pl.ds(j * _N, _N)], out_hbm.at[wid, j])

  return ek


_edge4 = _make_edge_kernel(4)
_edge2 = _make_edge_kernel(2)

_BN = 400


def _elu(x):
  return jnp.where(x > 0, x, jnp.exp(x) - 1.0)


def _mmT(wT, xT):
  m = wT.shape[0]
  n = xT.shape[1]

  def body(w_ref, x_ref, o_ref):
    o_ref[...] = jnp.dot(w_ref[...], x_ref[...],
                         preferred_element_type=jnp.float32)

  return pl.pallas_call(
      body,
      out_shape=jax.ShapeDtypeStruct((m, n), jnp.float32),
  )(wT, xT)


def _combine_mmT(aggT, w2T):
  m = w2T.shape[0]
  n = aggT.shape[1]

  def body(w_ref, a_ref, o_ref):
    o_ref[...] = jnp.dot(w_ref[...], _elu(a_ref[...]),
                         preferred_element_type=jnp.float32)

  return pl.pallas_call(
      body,
      out_shape=jax.ShapeDtypeStruct((m, n), jnp.float32),
  )(w2T, aggT)


def _eluT(aggT):
  m, n = aggT.shape

  def body(a_ref, o_ref):
    o_ref[...] = _elu(a_ref[...])

  return pl.pallas_call(
      body,
      out_shape=jax.ShapeDtypeStruct((m, n), jnp.float32),
  )(aggT)


def kernel(X_o, edge_index, edge_weight, W1, W2):
  src_r = edge_index[0]
  dst_r = edge_index[1]
  w_r = edge_weight

  hT = _mmT(W1.T, X_o.T)
  agg1 = _edge4(hT.reshape(_NW, 4, _N), src_r, dst_r, w_r)
  h2T = _combine_mmT(agg1.reshape(128, _N), W2.T)
  agg2 = _edge2(h2T.reshape(_NW, 2, _N), src_r, dst_r, w_r)
  return _eluT(agg2.reshape(64, _N)).T


# speedup vs baseline: 5.9247x; 5.9247x over previous
import functools

import jax
import jax.numpy as jnp
from jax import lax
from jax.experimental import pallas as pl
from jax.experimental.pallas import tpu as pltpu
from jax.experimental.pallas import tpu_sc as plsc

_N = 10000
_E = 320000
_CH = 80
_G = _CH // 16
_NW = 32
_EPW = _E // _NW
_NCH = _EPW // _CH
_RPT = 624
_REM = _N - 16 * _RPT


def _make_edge_kernel(D, tc_tiling=None):
  mesh = plsc.VectorSubcoreMesh(core_axis_name="c", subcore_axis_name="s")

  @functools.partial(
      pl.kernel,
      mesh=mesh,
      compiler_params=pltpu.CompilerParams(use_tc_tiling_on_sc=tc_tiling),
      out_type=jax.ShapeDtypeStruct((2, _N, D), jnp.float32),
      scratch_types=[
          pltpu.VMEM((_EPW,), jnp.float32),
          pltpu.VMEM((_CH,), jnp.int32),
          pltpu.VMEM((_CH,), jnp.int32),
          pltpu.VMEM((_CH,), jnp.int32),
          pltpu.VMEM((_CH,), jnp.int32),
          pltpu.VMEM((_CH, D), jnp.float32),
          pltpu.VMEM((_CH, D), jnp.float32),
          pltpu.VMEM_SHARED((_N, D), jnp.float32),
          pltpu.SemaphoreType.DMA,
          pltpu.SemaphoreType.DMA,
          pltpu.SemaphoreType.DMA,
          pltpu.SemaphoreType.DMA,
          pltpu.SemaphoreType.DMA,
          pltpu.SemaphoreType.DMA,
      ],
  )
  def ek(h_hbm, src_hbm, dst_hbm, w_hbm, out_hbm,
         w_v, sidx0, sidx1, didx0, didx1, rows0, rows1, acc,
         isem0, isem1, gsem0, gsem1, ssem0, ssem1):
    c = lax.axis_index("c")
    s = lax.axis_index("s")
    wid = s * 2 + c
    e0 = wid * _EPW

    sidx = (sidx0, sidx1)
    didx = (didx0, didx1)
    rows = (rows0, rows1)
    isem = (isem0, isem1)
    gsem = (gsem0, gsem1)
    ssem = (ssem0, ssem1)

    zv = jnp.zeros((16,), jnp.float32)

    def zrow(r, carry):
      for j in range(D // 16):
        rows0[r, pl.ds(j * 16, 16)] = zv
      return carry

    lax.fori_loop(0, _CH, zrow, 0)
    for t in range(_RPT // _CH):
      pltpu.sync_copy(rows0, acc.at[pl.ds(s * _RPT + t * _CH, _CH)])
    pltpu.sync_copy(rows0.at[pl.ds(0, _RPT % _CH)],
                    acc.at[pl.ds(s * _RPT + (_RPT // _CH) * _CH, _RPT % _CH)])

    @pl.when(s == 0)
    def _():
      pltpu.sync_copy(rows0.at[pl.ds(0, _REM)], acc.at[pl.ds(16 * _RPT, _REM)])

    pltpu.sync_copy(w_hbm.at[pl.ds(e0, _EPW)], w_v)
    plsc.subcore_barrier()

    def idx_start(ci, b):
      off = e0 + ci * _CH
      pltpu.async_copy(src_hbm.at[pl.ds(off, _CH)], sidx[b], isem[b])
      pltpu.async_copy(dst_hbm.at[pl.ds(off, _CH)], didx[b], isem[b])

    def idx_wait(ci, b):
      off = e0 + ci * _CH
      pltpu.make_async_copy(src_hbm.at[pl.ds(off, _CH)], sidx[b], isem[b]).wait()
      pltpu.make_async_copy(dst_hbm.at[pl.ds(off, _CH)], didx[b], isem[b]).wait()

    def gather_start(b):
      pltpu.async_copy(h_hbm.at[sidx[b]], rows[b], gsem[b])

    def gather_wait(b):
      pltpu.make_async_copy(h_hbm.at[sidx[b]], rows[b], gsem[b]).wait()

    def scat_start(b):
      pltpu.async_copy(rows[b], acc.at[didx[b]], ssem[b], add=True)

    def scat_wait(b):
      pltpu.make_async_copy(rows[b], acc.at[didx[b]], ssem[b]).wait()

    def scale(ci, b):
      rv = rows[b]
      for g in range(_G):
        wv16 = w_v[pl.ds(ci * _CH + g * 16, 16)]
        for l in range(16):
          wl = jnp.broadcast_to(wv16[l], (16,))
          r = g * 16 + l
          for j in range(D // 16):
            sl = pl.ds(j * 16, 16)
            rv[r, sl] = rv[r, sl] * wl

    idx_start(0, 0)
    idx_wait(0, 0)
    gather_start(0)

    def pair(ip, carry):
      for b in range(2):
        ci = 2 * ip + b
        nb = 1 - b

        @pl.when(ci > 0)
        def _():
          scat_wait(nb)

        idx_start(ci + 1, nb)
        gather_wait(b)
        idx_wait(ci + 1, nb)
        gather_start(nb)
        scale(ci, b)
        scat_start(b)
      return carry

    lax.fori_loop(0, (_NCH - 1) // 2, pair, 0)

    scat_wait(1)
    gather_wait(0)
    scale(_NCH - 1, 0)
    scat_start(0)
    scat_wait(0)

    plsc.subcore_barrier()
    pltpu.sync_copy(acc.at[pl.ds(s * _RPT, _RPT)],
                    out_hbm.at[c, pl.ds(s * _RPT, _RPT)])

    @pl.when(s == 0)
    def _():
      pltpu.sync_copy(acc.at[pl.ds(16 * _RPT, _REM)],
                      out_hbm.at[c, pl.ds(16 * _RPT, _REM)])

  return ek


_edge128 = _make_edge_kernel(128)
_edge64 = _make_edge_kernel(64, tc_tiling=False)

_BR = 400


def _elu(x):
  return jnp.where(x > 0, x, jnp.exp(x) - 1.0)


def _mm(x, w):
  n, k = x.shape
  m = w.shape[1]

  def body(x_ref, w_ref, o_ref):
    o_ref[...] = jnp.dot(x_ref[...], w_ref[...],
                         preferred_element_type=jnp.float32)

  return pl.pallas_call(
      body,
      grid=(n // _BR,),
      in_specs=[pl.BlockSpec((_BR, k), lambda i: (i, 0)),
                pl.BlockSpec((k, m), lambda i: (0, 0))],
      out_specs=pl.BlockSpec((_BR, m), lambda i: (i, 0)),
      out_shape=jax.ShapeDtypeStruct((n, m), jnp.float32),
  )(x, w)


def _combine_mm(p, w):
  _, n, k = p.shape
  m = w.shape[1]

  def body(p_ref, w_ref, o_ref):
    z = _elu(p_ref[0] + p_ref[1])
    o_ref[...] = jnp.dot(z, w_ref[...], preferred_element_type=jnp.float32)

  return pl.pallas_call(
      body,
      grid=(n // _BR,),
      in_specs=[pl.BlockSpec((2, _BR, k), lambda i: (0, i, 0)),
                pl.BlockSpec((k, m), lambda i: (0, 0))],
      out_specs=pl.BlockSpec((_BR, m), lambda i: (i, 0)),
      out_shape=jax.ShapeDtypeStruct((n, m), jnp.float32),
  )(p, w)


def _combine_elu(p):
  _, n, k = p.shape

  def body(p_ref, o_ref):
    o_ref[...] = _elu(p_ref[0] + p_ref[1])

  return pl.pallas_call(
      body,
      grid=(n // _BR,),
      in_specs=[pl.BlockSpec((2, _BR, k), lambda i: (0, i, 0))],
      out_specs=pl.BlockSpec((_BR, k), lambda i: (i, 0)),
      out_shape=jax.ShapeDtypeStruct((n, k), jnp.float32),
  )(p)


def kernel(X_o, edge_index, edge_weight, W1, W2):
  src_r = edge_index[0]
  dst_r = edge_index[1]
  w_r = edge_weight

  h = _mm(X_o, W1)
  p1 = _edge128(h, src_r, dst_r, w_r)
  h2 = _combine_mm(p1, W2)
  p2 = _edge64(h2, src_r, dst_r, w_r)
  return _combine_elu(p2)


# speedup vs baseline: 6.4942x; 1.0961x over previous
import functools

import jax
import jax.numpy as jnp
from jax import lax
from jax.experimental import pallas as pl
from jax.experimental.pallas import tpu as pltpu
from jax.experimental.pallas import tpu_sc as plsc

_N = 10000
_E = 320000
_CH = 80
_G = _CH // 16
_NW = 32
_EPW = _E // _NW
_NCH = _EPW // _CH
_RPT = 624
_REM = _N - 16 * _RPT


def _make_edge_kernel(D, tc_tiling=None):
  mesh = plsc.VectorSubcoreMesh(core_axis_name="c", subcore_axis_name="s")

  @functools.partial(
      pl.kernel,
      mesh=mesh,
      compiler_params=pltpu.CompilerParams(use_tc_tiling_on_sc=tc_tiling),
      out_type=jax.ShapeDtypeStruct((2, _N, D), jnp.float32),
      scratch_types=[
          pltpu.VMEM((_EPW,), jnp.float32),
          pltpu.VMEM((_EPW,), jnp.int32),
          pltpu.VMEM((_EPW,), jnp.int32),
          pltpu.VMEM((_CH, D), jnp.float32),
          pltpu.VMEM((_CH, D), jnp.float32),
          pltpu.VMEM_SHARED((_N, D), jnp.float32),
          pltpu.SemaphoreType.DMA,
          pltpu.SemaphoreType.DMA,
          pltpu.SemaphoreType.DMA,
          pltpu.SemaphoreType.DMA,
      ],
  )
  def ek(h_hbm, src_hbm, dst_hbm, w_hbm, out_hbm,
         w_v, src_v, dst_v, rows0, rows1, acc,
         gsem0, gsem1, ssem0, ssem1):
    c = lax.axis_index("c")
    s = lax.axis_index("s")
    wid = s * 2 + c
    e0 = wid * _EPW

    rows = (rows0, rows1)
    gsem = (gsem0, gsem1)
    ssem = (ssem0, ssem1)

    zv = jnp.zeros((16,), jnp.float32)

    def zrow(r, carry):
      for j in range(D // 16):
        rows0[r, pl.ds(j * 16, 16)] = zv
      return carry

    lax.fori_loop(0, _CH, zrow, 0)
    for t in range(_RPT // _CH):
      pltpu.sync_copy(rows0, acc.at[pl.ds(s * _RPT + t * _CH, _CH)])
    pltpu.sync_copy(rows0.at[pl.ds(0, _RPT % _CH)],
                    acc.at[pl.ds(s * _RPT + (_RPT // _CH) * _CH, _RPT % _CH)])

    @pl.when(s == 0)
    def _():
      pltpu.sync_copy(rows0.at[pl.ds(0, _REM)], acc.at[pl.ds(16 * _RPT, _REM)])

    pltpu.sync_copy(w_hbm.at[pl.ds(e0, _EPW)], w_v)
    pltpu.sync_copy(src_hbm.at[pl.ds(e0, _EPW)], src_v)
    pltpu.sync_copy(dst_hbm.at[pl.ds(e0, _EPW)], dst_v)
    plsc.subcore_barrier()

    def gather_start(ci, b):
      pltpu.async_copy(h_hbm.at[src_v.at[pl.ds(ci * _CH, _CH)]], rows[b],
                       gsem[b])

    def gather_wait(ci, b):
      pltpu.make_async_copy(h_hbm.at[src_v.at[pl.ds(ci * _CH, _CH)]], rows[b],
                            gsem[b]).wait()

    def scat_start(ci, b):
      pltpu.async_copy(rows[b], acc.at[dst_v.at[pl.ds(ci * _CH, _CH)]],
                       ssem[b], add=True)

    def scat_wait(ci, b):
      pltpu.make_async_copy(rows[b], acc.at[dst_v.at[pl.ds(ci * _CH, _CH)]],
                            ssem[b]).wait()

    def scale(ci, b):
      rv = rows[b]
      for g in range(_G):
        wv16 = w_v[pl.ds(ci * _CH + g * 16, 16)]
        for l in range(16):
          wl = jnp.broadcast_to(wv16[l], (16,))
          r = g * 16 + l
          for j in range(D // 16):
            sl = pl.ds(j * 16, 16)
            rv[r, sl] = rv[r, sl] * wl

    gather_start(0, 0)

    def pair(ip, carry):
      for b in range(2):
        ci = 2 * ip + b
        nb = 1 - b

        @pl.when(ci > 0)
        def _():
          scat_wait(ci - 1, nb)

        gather_wait(ci, b)
        gather_start(ci + 1, nb)
        scale(ci, b)
        scat_start(ci, b)
      return carry

    lax.fori_loop(0, (_NCH - 1) // 2, pair, 0)

    scat_wait(_NCH - 2, 1)
    gather_wait(_NCH - 1, 0)
    scale(_NCH - 1, 0)
    scat_start(_NCH - 1, 0)
    scat_wait(_NCH - 1, 0)

    plsc.subcore_barrier()
    pltpu.sync_copy(acc.at[pl.ds(s * _RPT, _RPT)],
                    out_hbm.at[c, pl.ds(s * _RPT, _RPT)])

    @pl.when(s == 0)
    def _():
      pltpu.sync_copy(acc.at[pl.ds(16 * _RPT, _REM)],
                      out_hbm.at[c, pl.ds(16 * _RPT, _REM)])

  return ek


_edge128 = _make_edge_kernel(128)
_edge64 = _make_edge_kernel(64, tc_tiling=False)

_BR = 400


def _elu(x):
  return jnp.where(x > 0, x, jnp.exp(x) - 1.0)


def _mm(x, w):
  n, k = x.shape
  m = w.shape[1]

  def body(x_ref, w_ref, o_ref):
    o_ref[...] = jnp.dot(x_ref[...], w_ref[...],
                         preferred_element_type=jnp.float32)

  return pl.pallas_call(
      body,
      grid=(n // _BR,),
      in_specs=[pl.BlockSpec((_BR, k), lambda i: (i, 0)),
                pl.BlockSpec((k, m), lambda i: (0, 0))],
      out_specs=pl.BlockSpec((_BR, m), lambda i: (i, 0)),
      out_shape=jax.ShapeDtypeStruct((n, m), jnp.float32),
  )(x, w)


def _combine_mm(p, w):
  _, n, k = p.shape
  m = w.shape[1]

  def body(p_ref, w_ref, o_ref):
    z = _elu(p_ref[0] + p_ref[1])
    o_ref[...] = jnp.dot(z, w_ref[...], preferred_element_type=jnp.float32)

  return pl.pallas_call(
      body,
      grid=(n // _BR,),
      in_specs=[pl.BlockSpec((2, _BR, k), lambda i: (0, i, 0)),
                pl.BlockSpec((k, m), lambda i: (0, 0))],
      out_specs=pl.BlockSpec((_BR, m), lambda i: (i, 0)),
      out_shape=jax.ShapeDtypeStruct((n, m), jnp.float32),
  )(p, w)


def _combine_elu(p):
  _, n, k = p.shape

  def body(p_ref, o_ref):
    o_ref[...] = _elu(p_ref[0] + p_ref[1])

  return pl.pallas_call(
      body,
      grid=(n // _BR,),
      in_specs=[pl.BlockSpec((2, _BR, k), lambda i: (0, i, 0))],
      out_specs=pl.BlockSpec((_BR, k), lambda i: (i, 0)),
      out_shape=jax.ShapeDtypeStruct((n, k), jnp.float32),
  )(p)


def kernel(X_o, edge_index, edge_weight, W1, W2):
  src_r = edge_index[0]
  dst_r = edge_index[1]
  w_r = edge_weight

  h = _mm(X_o, W1)
  p1 = _edge128(h, src_r, dst_r, w_r)
  h2 = _combine_mm(p1, W2)
  p2 = _edge64(h2, src_r, dst_r, w_r)
  return _combine_elu(p2)
